# baseline (speedup 1.0000x reference)
import functools

import jax
import jax.numpy as jnp
from jax import lax
from jax.experimental import pallas as pl
from jax.experimental.pallas import tpu as pltpu
from jax.experimental.pallas import tpu_sc as plsc

N_TOK = 32768
D_EMB = 512
S_DIM = 256

N_SPLIT = 1
SPLIT = N_TOK // N_SPLIT

NC = 2
NS = 16
NW = NC * NS
B_PER_W = SPLIT // NW
CH = 64
N_CHUNK = B_PER_W // CH

_sc_mesh = plsc.VectorSubcoreMesh(core_axis_name="c", subcore_axis_name="s")


@functools.partial(
    pl.kernel,
    out_type=jax.ShapeDtypeStruct((SPLIT, D_EMB), jnp.float32),
    mesh=_sc_mesh,
    scratch_types=[
        pltpu.VMEM((B_PER_W,), jnp.int32),
        pltpu.VMEM((CH, D_EMB), jnp.float32),
        pltpu.VMEM((CH, D_EMB), jnp.float32),
        pltpu.SemaphoreType.DMA,
        pltpu.SemaphoreType.DMA,
    ],
)
def _sc_gather(word_hbm, table_hbm, out_hbm, idx_v, rows_a, rows_b, gsem, ssem):
    wid = lax.axis_index("s") * NC + lax.axis_index("c")
    base = wid * B_PER_W
    pltpu.sync_copy(word_hbm.at[pl.ds(base, B_PER_W)], idx_v)
    bufs = (rows_a, rows_b)
    gathers = [None] * N_CHUNK
    stores = [None] * N_CHUNK
    for c in range(min(2, N_CHUNK)):
        gathers[c] = pltpu.async_copy(
            table_hbm.at[idx_v.at[pl.ds(c * CH, CH)]], bufs[c % 2], gsem
        )
    for c in range(N_CHUNK):
        buf = bufs[c % 2]
        gathers[c].wait()
        stores[c] = pltpu.async_copy(
            buf, out_hbm.at[pl.ds(base + c * CH, CH)], ssem
        )
        if c + 2 < N_CHUNK:
            stores[c].wait()
            gathers[c + 2] = pltpu.async_copy(
                table_hbm.at[idx_v.at[pl.ds((c + 2) * CH, CH)]], buf, gsem
            )
    stores[N_CHUNK - 2].wait()
    stores[N_CHUNK - 1].wait()


BM = 2048
BPS = SPLIT // BM


KS = D_EMB // 2


def _mlp_body(x1_ref, x2_ref, w1_ref, w2_ref, b_ref, o_ref):
    acc = jnp.dot(
        x1_ref[...].astype(jnp.bfloat16),
        w1_ref[...].astype(jnp.bfloat16),
        preferred_element_type=jnp.float32,
    )
    acc = acc + jnp.dot(
        x2_ref[...].astype(jnp.bfloat16),
        w2_ref[...].astype(jnp.bfloat16),
        preferred_element_type=jnp.float32,
    )
    o_ref[...] = jnp.tanh(acc + b_ref[...])


def _mlp_body_alias(full_ref, x1_ref, x2_ref, w1_ref, w2_ref, b_ref, o_ref):
    del full_ref
    _mlp_body(x1_ref, x2_ref, w1_ref, w2_ref, b_ref, o_ref)


def _tc_mlp_chunk(ci, emb_chunk, W1, b1r, full=None):
    in_specs = [
        pl.BlockSpec((BM, KS), lambda i: (i, 0)),
        pl.BlockSpec((BM, KS), lambda i: (i, 1)),
        pl.BlockSpec((KS, S_DIM), lambda i: (0, 0)),
        pl.BlockSpec((KS, S_DIM), lambda i: (1, 0)),
        pl.BlockSpec((1, S_DIM), lambda i: (0, 0)),
    ]
    body = _mlp_body
    args = (emb_chunk, emb_chunk, W1, W1, b1r)
    io_aliases = {}
    if full is not None:
        in_specs = [pl.BlockSpec(memory_space=pl.ANY)] + in_specs
        body = _mlp_body_alias
        args = (full,) + args
        io_aliases = {0: 0}
    return pl.pallas_call(
        body,
        grid=(BPS,),
        in_specs=in_specs,
        out_specs=pl.BlockSpec((BM, S_DIM), lambda i, ci=ci: (ci * BPS + i, 0)),
        out_shape=jax.ShapeDtypeStruct((N_TOK, S_DIM), jnp.float32),
        input_output_aliases=io_aliases,
    )(*args)


def kernel(word, word_embeddings, grammar_preterminates, W1, b1):
    del grammar_preterminates
    word = word.astype(jnp.int32)
    b1r = b1.reshape(1, S_DIM)
    embs = [
        _sc_gather(word[ci * SPLIT:(ci + 1) * SPLIT], word_embeddings)
        for ci in range(N_SPLIT)
    ]
    full = _tc_mlp_chunk(0, embs[0], W1, b1r)
    for ci in range(1, N_SPLIT):
        full = _tc_mlp_chunk(ci, embs[ci], W1, b1r, full)
    return full


# speedup vs baseline: 1.0277x; 1.0277x over previous
import functools

import jax
import jax.numpy as jnp
from jax import lax
from jax.experimental import pallas as pl
from jax.experimental.pallas import tpu as pltpu
from jax.experimental.pallas import tpu_sc as plsc

N_TOK = 32768
D_EMB = 512
S_DIM = 256

N_SPLIT = 1
SPLIT = N_TOK // N_SPLIT

NC = 2
NS = 16
NW = NC * NS
B_PER_W = SPLIT // NW
CH = 64
N_CHUNK = B_PER_W // CH

_sc_mesh = plsc.VectorSubcoreMesh(core_axis_name="c", subcore_axis_name="s")


@functools.partial(
    pl.kernel,
    out_type=jax.ShapeDtypeStruct((SPLIT, D_EMB), jnp.float32),
    mesh=_sc_mesh,
    scratch_types=[
        pltpu.VMEM((B_PER_W,), jnp.int32),
        pltpu.VMEM((CH, D_EMB), jnp.float32),
        pltpu.VMEM((CH, D_EMB), jnp.float32),
        pltpu.SemaphoreType.DMA,
        pltpu.SemaphoreType.DMA,
    ],
)
def _sc_gather(word_hbm, table_hbm, out_hbm, idx_v, rows_a, rows_b, gsem, ssem):
    wid = lax.axis_index("s") * NC + lax.axis_index("c")
    base = wid * B_PER_W
    pltpu.sync_copy(word_hbm.at[pl.ds(base, B_PER_W)], idx_v)
    bufs = (rows_a, rows_b)
    gathers = [None] * N_CHUNK
    stores = [None] * N_CHUNK
    for c in range(min(2, N_CHUNK)):
        gathers[c] = pltpu.async_copy(
            table_hbm.at[idx_v.at[pl.ds(c * CH, CH)]], bufs[c % 2], gsem
        )
    for c in range(N_CHUNK):
        buf = bufs[c % 2]
        gathers[c].wait()
        stores[c] = pltpu.async_copy(
            buf, out_hbm.at[pl.ds(base + c * CH, CH)], ssem
        )
        if c + 2 < N_CHUNK:
            stores[c].wait()
            gathers[c + 2] = pltpu.async_copy(
                table_hbm.at[idx_v.at[pl.ds((c + 2) * CH, CH)]], buf, gsem
            )
    stores[N_CHUNK - 2].wait()
    stores[N_CHUNK - 1].wait()


BM = 4096
BPS = SPLIT // BM


KS = D_EMB // 2


def _mlp_body(x1_ref, x2_ref, w1_ref, w2_ref, b_ref, o_ref):
    acc = jnp.dot(
        x1_ref[...].astype(jnp.bfloat16),
        w1_ref[...].astype(jnp.bfloat16),
        preferred_element_type=jnp.float32,
    )
    acc = acc + jnp.dot(
        x2_ref[...].astype(jnp.bfloat16),
        w2_ref[...].astype(jnp.bfloat16),
        preferred_element_type=jnp.float32,
    )
    o_ref[...] = jnp.tanh(acc + b_ref[...])


def _mlp_body_alias(full_ref, x1_ref, x2_ref, w1_ref, w2_ref, b_ref, o_ref):
    del full_ref
    _mlp_body(x1_ref, x2_ref, w1_ref, w2_ref, b_ref, o_ref)


def _tc_mlp_chunk(ci, emb_chunk, W1, b1r, full=None):
    in_specs = [
        pl.BlockSpec((BM, KS), lambda i: (i, 0)),
        pl.BlockSpec((BM, KS), lambda i: (i, 1)),
        pl.BlockSpec((KS, S_DIM), lambda i: (0, 0)),
        pl.BlockSpec((KS, S_DIM), lambda i: (1, 0)),
        pl.BlockSpec((1, S_DIM), lambda i: (0, 0)),
    ]
    body = _mlp_body
    args = (emb_chunk, emb_chunk, W1, W1, b1r)
    io_aliases = {}
    if full is not None:
        in_specs = [pl.BlockSpec(memory_space=pl.ANY)] + in_specs
        body = _mlp_body_alias
        args = (full,) + args
        io_aliases = {0: 0}
    return pl.pallas_call(
        body,
        grid=(BPS,),
        in_specs=in_specs,
        out_specs=pl.BlockSpec((BM, S_DIM), lambda i, ci=ci: (ci * BPS + i, 0)),
        out_shape=jax.ShapeDtypeStruct((N_TOK, S_DIM), jnp.float32),
        input_output_aliases=io_aliases,
    )(*args)


def kernel(word, word_embeddings, grammar_preterminates, W1, b1):
    del grammar_preterminates
    word = word.astype(jnp.int32)
    b1r = b1.reshape(1, S_DIM)
    embs = [
        _sc_gather(word[ci * SPLIT:(ci + 1) * SPLIT], word_embeddings)
        for ci in range(N_SPLIT)
    ]
    full = _tc_mlp_chunk(0, embs[0], W1, b1r)
    for ci in range(1, N_SPLIT):
        full = _tc_mlp_chunk(ci, embs[ci], W1, b1r, full)
    return full
